# baseline (speedup 1.0000x reference)
import jax
import jax.numpy as jnp
import numpy as np
from jax.experimental import pallas as pl
from jax.experimental.pallas import tpu as pltpu

_ANCHORS = np.array(
    [[[10., 13.], [16., 30.], [33., 23.]],
     [[30., 61.], [62., 45.], [59., 119.]],
     [[116., 90.], [156., 198.], [373., 326.]]], dtype=np.float32)
_ANCHOR_T = 4.0
_BALANCE = (4.0, 1.0, 0.4)
_BOX_W, _CLS_W, _OBJ_W = 0.05, 0.5, 1.0
_NC = 7
_NA = 3


def _softplus(x):
    return jnp.maximum(x, 0.0) + jnp.log1p(jnp.exp(-jnp.abs(x)))


def _obj_sum_kernel(p_ref, o_ref):
    @pl.when(pl.program_id(0) == 0)
    def _init():
        o_ref[0, 0] = 0.0

    x = p_ref[...]
    o_ref[0, 0] += jnp.sum(_softplus(x))


def _obj_softplus_sum(pred):
    bs, _, gh, gw = pred.shape
    out = pl.pallas_call(
        _obj_sum_kernel,
        grid=(_NA,),
        in_specs=[pl.BlockSpec((bs, 1, gh, gw), lambda a: (0, 4 + 12 * a, 0, 0))],
        out_specs=pl.BlockSpec((1, 1), lambda a: (0, 0), memory_space=pltpu.SMEM),
        out_shape=jax.ShapeDtypeStruct((1, 1), jnp.float32),
    )(pred)
    return out[0, 0]


def _sparse_kernel(ps_ref, bt_ref, cls_ref, mf_ref, wd_ref, o_ref):
    nl = ps_ref.shape[0]
    m = ps_ref.shape[2]
    for l in range(nl):
        ps = ps_ref[l]
        bt = bt_ref[l]
        cls = cls_ref[l]
        mf = mf_ref[l]
        wd = wd_ref[l]

        b1x, b1y, b1w, b1h = ps[0:1], ps[1:2], ps[2:3], ps[3:4]
        b2x, b2y, b2w, b2h = bt[0:1], bt[1:2], bt[2:3], bt[3:4]
        b1x1 = b1x - b1w * 0.5
        b1x2 = b1x + b1w * 0.5
        b1y1 = b1y - b1h * 0.5
        b1y2 = b1y + b1h * 0.5
        b2x1 = b2x - b2w * 0.5
        b2x2 = b2x + b2w * 0.5
        b2y1 = b2y - b2h * 0.5
        b2y2 = b2y + b2h * 0.5
        iw = jnp.maximum(jnp.minimum(b1x2, b2x2) - jnp.maximum(b1x1, b2x1), 0.0)
        ih = jnp.maximum(jnp.minimum(b1y2, b2y2) - jnp.maximum(b1y1, b2y1), 0.0)
        inter = iw * ih
        union = ((b1x2 - b1x1) * (b1y2 - b1y1)
                 + (b2x2 - b2x1) * (b2y2 - b2y1) - inter + 1e-7)
        iou = inter / union
        box_sum = jnp.sum(mf * (1.0 - iou))

        xc = ps[5:12]
        tc = (jax.lax.broadcasted_iota(jnp.int32, (_NC, m), 0).astype(jnp.float32)
              == cls)
        tc = tc.astype(jnp.float32)
        bce = jnp.maximum(xc, 0.0) - xc * tc + jnp.log1p(jnp.exp(-jnp.abs(xc)))
        cls_sum = jnp.sum(mf * bce)

        obj_corr = jnp.sum(wd * ps[4:5])
        cnt = jnp.sum(mf)

        o_ref[l, 0] = box_sum
        o_ref[l, 1] = cls_sum
        o_ref[l, 2] = obj_corr
        o_ref[l, 3] = cnt


def _sparse_call(psel_t, box_t, cls_r, mf_r, wd_r):
    nl, _, m = psel_t.shape
    return pl.pallas_call(
        _sparse_kernel,
        out_specs=pl.BlockSpec(memory_space=pltpu.SMEM),
        out_shape=jax.ShapeDtypeStruct((nl, 4), jnp.float32),
    )(psel_t, box_t, cls_r, mf_r, wd_r)


def _prep_level(bt, bs, gh, gw, layer):
    nt = bt.shape[0]
    valid = jnp.isfinite(bt).all(1) & (bt[:, 1:5] > 0).all(1)
    av = jnp.asarray(_ANCHORS[layer])
    r = bt[:, 3:5][:, None, :] / av[None, :, :]
    j = jnp.maximum(r, 1.0 / r).max(2) < _ANCHOR_T
    mask = (valid[:, None] & j).reshape(-1)
    gx = bt[:, 2] * gw
    gy = bt[:, 3] * gh
    gi = jnp.clip(jnp.floor(gx).astype(jnp.int32), 0, gw - 1)
    gj = jnp.clip(jnp.floor(gy).astype(jnp.int32), 0, gh - 1)
    b = jnp.clip(bt[:, 0].astype(jnp.int32), 0, bs - 1)

    box_t = jnp.stack([
        jnp.broadcast_to((gx - gi)[:, None], (nt, _NA)),
        jnp.broadcast_to((gy - gj)[:, None], (nt, _NA)),
        bt[:, 3][:, None] / av[None, :, 0],
        bt[:, 4][:, None] / av[None, :, 1],
    ], axis=2).astype(jnp.float32).reshape(-1, 4)
    cls_t = jnp.clip(bt[:, 1].astype(jnp.int32), 0, _NC - 1)

    a_b = jnp.broadcast_to(jnp.arange(_NA, dtype=jnp.int32)[None, :], (nt, _NA)).reshape(-1)
    b_b = jnp.broadcast_to(b[:, None], (nt, _NA)).reshape(-1)
    gj_b = jnp.broadcast_to(gj[:, None], (nt, _NA)).reshape(-1)
    gi_b = jnp.broadcast_to(gi[:, None], (nt, _NA)).reshape(-1)
    cls_b = jnp.broadcast_to(cls_t[:, None], (nt, _NA)).reshape(-1)

    hw = gh * gw
    cell = ((b_b * _NA + a_b) * gh + gj_b) * gw + gi_b
    chan0 = (b_b * 36 + a_b * 12) * hw + gj_b * gw + gi_b
    idx = chan0[:, None] + (jnp.arange(12, dtype=jnp.int32) * hw)[None, :]

    m = mask.shape[0]
    mf = mask.astype(jnp.float32)
    big = bs * _NA * hw
    ids = jnp.where(mask, cell, big)
    order = jnp.argsort(ids)
    ids_s = ids[order]
    prev = jnp.concatenate([jnp.full((1,), -1, ids_s.dtype), ids_s[:-1]])
    uniq = (ids_s != prev) & (ids_s < big)
    wd = jnp.zeros((m,), jnp.float32).at[order].set(uniq.astype(jnp.float32))
    return idx, box_t, cls_b, mf, wd


def _yolo_loss(p3, p4, p5, targets):
    preds = [p3, p4, p5]
    bt = targets.astype(jnp.float32)

    obj_sums = []
    psel_ts, box_ts, cls_rs, mf_rs, wd_rs, inv_ns = [], [], [], [], [], []
    for i, pred in enumerate(preds):
        bs, _, gh, gw = pred.shape
        idx, box_t, cls_b, mf, wd = _prep_level(bt, bs, gh, gw, i)
        psel = pred.reshape(-1)[idx]
        psel_ts.append(psel.T)
        box_ts.append(box_t.T)
        cls_rs.append(cls_b.astype(jnp.float32)[None, :])
        mf_rs.append(mf[None, :])
        wd_rs.append(wd[None, :])
        inv_ns.append(1.0 / (bs * _NA * gh * gw))
        obj_sums.append(_obj_softplus_sum(pred))

    out = _sparse_call(jnp.stack(psel_ts), jnp.stack(box_ts),
                       jnp.stack(cls_rs), jnp.stack(mf_rs), jnp.stack(wd_rs))
    box_sums, cls_sums, corrs, cnts = out[:, 0], out[:, 1], out[:, 2], out[:, 3]

    safe = jnp.maximum(cnts, 1.0)
    has = (cnts > 0).astype(jnp.float32)
    nlay = jnp.maximum(has.sum(), 1.0)
    obj_l = (jnp.stack(obj_sums) - corrs) * jnp.asarray(inv_ns, jnp.float32)
    obj_l = obj_l * jnp.asarray(_BALANCE, jnp.float32)

    lbox = (jnp.sum((box_sums / safe) * has) / nlay) * _BOX_W
    lobj = jnp.mean(obj_l) * _OBJ_W
    lcls = (jnp.sum((cls_sums / (safe * _NC)) * has) / nlay) * _CLS_W
    loss = lbox + lobj + lcls
    return loss, lbox, lobj, lcls


_jitted = jax.jit(_yolo_loss)


def kernel(p3, p4, p5, targets):
    return _jitted(p3, p4, p5, targets)


# speedup vs baseline: 1.0455x; 1.0455x over previous
import jax
import jax.numpy as jnp
import numpy as np
from jax.experimental import pallas as pl
from jax.experimental.pallas import tpu as pltpu

_ANCHORS = np.array(
    [[[10., 13.], [16., 30.], [33., 23.]],
     [[30., 61.], [62., 45.], [59., 119.]],
     [[116., 90.], [156., 198.], [373., 326.]]], dtype=np.float32)
_ANCHOR_T = 4.0
_BALANCE = (4.0, 1.0, 0.4)
_BOX_W, _CLS_W, _OBJ_W = 0.05, 0.5, 1.0
_NC = 7
_NA = 3


def _softplus(x):
    return jnp.maximum(x, 0.0) + jnp.log1p(jnp.exp(-jnp.abs(x)))


def _obj_sum_kernel(p_ref, m_ref, o_ref):
    @pl.when(pl.program_id(0) == 0)
    def _init():
        o_ref[0, 0] = 0.0
        o_ref[0, 1] = 0.0

    x = p_ref[...]
    o_ref[0, 0] += jnp.sum(_softplus(x))
    o_ref[0, 1] += jnp.sum(m_ref[...] * x)


def _obj_softplus_sum(pred, tmask):
    bs, _, gh, gw = pred.shape
    out = pl.pallas_call(
        _obj_sum_kernel,
        grid=(_NA,),
        in_specs=[
            pl.BlockSpec((bs, 1, gh, gw), lambda a: (0, 4 + 12 * a, 0, 0)),
            pl.BlockSpec((bs, 1, gh, gw), lambda a: (0, a, 0, 0)),
        ],
        out_specs=pl.BlockSpec((1, 2), lambda a: (0, 0), memory_space=pltpu.SMEM),
        out_shape=jax.ShapeDtypeStruct((1, 2), jnp.float32),
    )(pred, tmask)
    return out[0, 0], out[0, 1]


def _sparse_kernel(ps_ref, bt_ref, cls_ref, mf_ref, o_ref):
    nl = ps_ref.shape[0]
    m = ps_ref.shape[2]
    for l in range(nl):
        ps = ps_ref[l]
        bt = bt_ref[l]
        cls = cls_ref[l]
        mf = mf_ref[l]

        b1x, b1y, b1w, b1h = ps[0:1], ps[1:2], ps[2:3], ps[3:4]
        b2x, b2y, b2w, b2h = bt[0:1], bt[1:2], bt[2:3], bt[3:4]
        b1x1 = b1x - b1w * 0.5
        b1x2 = b1x + b1w * 0.5
        b1y1 = b1y - b1h * 0.5
        b1y2 = b1y + b1h * 0.5
        b2x1 = b2x - b2w * 0.5
        b2x2 = b2x + b2w * 0.5
        b2y1 = b2y - b2h * 0.5
        b2y2 = b2y + b2h * 0.5
        iw = jnp.maximum(jnp.minimum(b1x2, b2x2) - jnp.maximum(b1x1, b2x1), 0.0)
        ih = jnp.maximum(jnp.minimum(b1y2, b2y2) - jnp.maximum(b1y1, b2y1), 0.0)
        inter = iw * ih
        union = ((b1x2 - b1x1) * (b1y2 - b1y1)
                 + (b2x2 - b2x1) * (b2y2 - b2y1) - inter + 1e-7)
        iou = inter / union
        box_sum = jnp.sum(mf * (1.0 - iou))

        xc = ps[5:12]
        tc = (jax.lax.broadcasted_iota(jnp.int32, (_NC, m), 0).astype(jnp.float32)
              == cls)
        tc = tc.astype(jnp.float32)
        bce = jnp.maximum(xc, 0.0) - xc * tc + jnp.log1p(jnp.exp(-jnp.abs(xc)))
        cls_sum = jnp.sum(mf * bce)

        cnt = jnp.sum(mf)

        o_ref[l, 0] = box_sum
        o_ref[l, 1] = cls_sum
        o_ref[l, 2] = cnt


def _sparse_call(psel_t, box_t, cls_r, mf_r):
    nl, _, m = psel_t.shape
    return pl.pallas_call(
        _sparse_kernel,
        out_specs=pl.BlockSpec(memory_space=pltpu.SMEM),
        out_shape=jax.ShapeDtypeStruct((nl, 3), jnp.float32),
    )(psel_t, box_t, cls_r, mf_r)


def _prep_level(bt, bs, gh, gw, layer):
    nt = bt.shape[0]
    valid = jnp.isfinite(bt).all(1) & (bt[:, 1:5] > 0).all(1)
    av = jnp.asarray(_ANCHORS[layer])
    r = bt[:, 3:5][:, None, :] / av[None, :, :]
    j = jnp.maximum(r, 1.0 / r).max(2) < _ANCHOR_T
    mask = (valid[:, None] & j).reshape(-1)
    gx = bt[:, 2] * gw
    gy = bt[:, 3] * gh
    gi = jnp.clip(jnp.floor(gx).astype(jnp.int32), 0, gw - 1)
    gj = jnp.clip(jnp.floor(gy).astype(jnp.int32), 0, gh - 1)
    b = jnp.clip(bt[:, 0].astype(jnp.int32), 0, bs - 1)

    box_t = jnp.stack([
        jnp.broadcast_to((gx - gi)[:, None], (nt, _NA)),
        jnp.broadcast_to((gy - gj)[:, None], (nt, _NA)),
        bt[:, 3][:, None] / av[None, :, 0],
        bt[:, 4][:, None] / av[None, :, 1],
    ], axis=2).astype(jnp.float32).reshape(-1, 4)
    cls_t = jnp.clip(bt[:, 1].astype(jnp.int32), 0, _NC - 1)

    a_b = jnp.broadcast_to(jnp.arange(_NA, dtype=jnp.int32)[None, :], (nt, _NA)).reshape(-1)
    b_b = jnp.broadcast_to(b[:, None], (nt, _NA)).reshape(-1)
    gj_b = jnp.broadcast_to(gj[:, None], (nt, _NA)).reshape(-1)
    gi_b = jnp.broadcast_to(gi[:, None], (nt, _NA)).reshape(-1)
    cls_b = jnp.broadcast_to(cls_t[:, None], (nt, _NA)).reshape(-1)

    hw = gh * gw
    cell = ((b_b * _NA + a_b) * gh + gj_b) * gw + gi_b
    chan0 = (b_b * 36 + a_b * 12) * hw + gj_b * gw + gi_b
    idx = chan0[:, None] + (jnp.arange(12, dtype=jnp.int32) * hw)[None, :]

    mf = mask.astype(jnp.float32)
    big = bs * _NA * hw
    ids = jnp.where(mask, cell, big)
    tmask = (jnp.zeros((big,), jnp.float32)
             .at[ids].set(1.0, mode="drop")
             .reshape(bs, _NA, gh, gw))
    return idx, box_t, cls_b, mf, tmask


def _yolo_loss(p3, p4, p5, targets):
    preds = [p3, p4, p5]
    bt = targets.astype(jnp.float32)

    obj_sums, corr_sums = [], []
    psel_ts, box_ts, cls_rs, mf_rs, inv_ns = [], [], [], [], []
    for i, pred in enumerate(preds):
        bs, _, gh, gw = pred.shape
        idx, box_t, cls_b, mf, tmask = _prep_level(bt, bs, gh, gw, i)
        psel = pred.reshape(-1)[idx]
        psel_ts.append(psel.T)
        box_ts.append(box_t.T)
        cls_rs.append(cls_b.astype(jnp.float32)[None, :])
        mf_rs.append(mf[None, :])
        inv_ns.append(1.0 / (bs * _NA * gh * gw))
        s, c = _obj_softplus_sum(pred, tmask)
        obj_sums.append(s)
        corr_sums.append(c)

    out = _sparse_call(jnp.stack(psel_ts), jnp.stack(box_ts),
                       jnp.stack(cls_rs), jnp.stack(mf_rs))
    box_sums, cls_sums, cnts = out[:, 0], out[:, 1], out[:, 2]
    corrs = jnp.stack(corr_sums)

    safe = jnp.maximum(cnts, 1.0)
    has = (cnts > 0).astype(jnp.float32)
    nlay = jnp.maximum(has.sum(), 1.0)
    obj_l = (jnp.stack(obj_sums) - corrs) * jnp.asarray(inv_ns, jnp.float32)
    obj_l = obj_l * jnp.asarray(_BALANCE, jnp.float32)

    lbox = (jnp.sum((box_sums / safe) * has) / nlay) * _BOX_W
    lobj = jnp.mean(obj_l) * _OBJ_W
    lcls = (jnp.sum((cls_sums / (safe * _NC)) * has) / nlay) * _CLS_W
    loss = lbox + lobj + lcls
    return loss, lbox, lobj, lcls


_jitted = jax.jit(_yolo_loss)


def kernel(p3, p4, p5, targets):
    return _jitted(p3, p4, p5, targets)


# speedup vs baseline: 1.0820x; 1.0349x over previous
import jax
import jax.numpy as jnp
import numpy as np
from jax.experimental import pallas as pl
from jax.experimental.pallas import tpu as pltpu

_ANCHORS = np.array(
    [[[10., 13.], [16., 30.], [33., 23.]],
     [[30., 61.], [62., 45.], [59., 119.]],
     [[116., 90.], [156., 198.], [373., 326.]]], dtype=np.float32)
_ANCHOR_T = 4.0
_BALANCE = (4.0, 1.0, 0.4)
_BOX_W, _CLS_W, _OBJ_W = 0.05, 0.5, 1.0
_NC = 7
_NA = 3


def _softplus(x):
    return jnp.maximum(x, 0.0) + jnp.log1p(jnp.exp(-jnp.abs(x)))


def _fused_kernel(p3r, p4r, p5r, m3r, m4r, m5r, ps_ref, bt_ref, cls_ref,
                  mf_ref, o_ref):
    a = pl.program_id(0)

    @pl.when(a == 0)
    def _init_and_sparse():
        for l in range(3):
            o_ref[l, 3] = 0.0
            o_ref[l, 4] = 0.0
        _sparse_body(ps_ref, bt_ref, cls_ref, mf_ref, o_ref)

    for l, (pr, mr) in enumerate(((p3r, m3r), (p4r, m4r), (p5r, m5r))):
        x = pr[...]
        o_ref[l, 3] += jnp.sum(_softplus(x))
        o_ref[l, 4] += jnp.sum(mr[...] * x)


def _sparse_body(ps_ref, bt_ref, cls_ref, mf_ref, o_ref):
    nl = ps_ref.shape[0]
    m = ps_ref.shape[2]
    for l in range(nl):
        ps = ps_ref[l]
        bt = bt_ref[l]
        cls = cls_ref[l]
        mf = mf_ref[l]

        b1x, b1y, b1w, b1h = ps[0:1], ps[1:2], ps[2:3], ps[3:4]
        b2x, b2y, b2w, b2h = bt[0:1], bt[1:2], bt[2:3], bt[3:4]
        b1x1 = b1x - b1w * 0.5
        b1x2 = b1x + b1w * 0.5
        b1y1 = b1y - b1h * 0.5
        b1y2 = b1y + b1h * 0.5
        b2x1 = b2x - b2w * 0.5
        b2x2 = b2x + b2w * 0.5
        b2y1 = b2y - b2h * 0.5
        b2y2 = b2y + b2h * 0.5
        iw = jnp.maximum(jnp.minimum(b1x2, b2x2) - jnp.maximum(b1x1, b2x1), 0.0)
        ih = jnp.maximum(jnp.minimum(b1y2, b2y2) - jnp.maximum(b1y1, b2y1), 0.0)
        inter = iw * ih
        union = ((b1x2 - b1x1) * (b1y2 - b1y1)
                 + (b2x2 - b2x1) * (b2y2 - b2y1) - inter + 1e-7)
        iou = inter / union
        box_sum = jnp.sum(mf * (1.0 - iou))

        xc = ps[5:12]
        tc = (jax.lax.broadcasted_iota(jnp.int32, (_NC, m), 0).astype(jnp.float32)
              == cls)
        tc = tc.astype(jnp.float32)
        bce = jnp.maximum(xc, 0.0) - xc * tc + jnp.log1p(jnp.exp(-jnp.abs(xc)))
        cls_sum = jnp.sum(mf * bce)

        cnt = jnp.sum(mf)

        o_ref[l, 0] = box_sum
        o_ref[l, 1] = cls_sum
        o_ref[l, 2] = cnt


def _fused_call(preds, tmasks, psel_t, box_t, cls_r, mf_r):
    dense_specs = []
    for p in preds:
        bs, _, gh, gw = p.shape
        dense_specs.append(
            pl.BlockSpec((bs, 1, gh, gw), lambda a: (0, 4 + 12 * a, 0, 0)))
    for t in tmasks:
        bs, _, gh, gw = t.shape
        dense_specs.append(pl.BlockSpec((bs, 1, gh, gw), lambda a: (0, a, 0, 0)))
    full3 = [pl.BlockSpec(x.shape, lambda a: (0, 0, 0))
             for x in (psel_t, box_t, cls_r, mf_r)]
    return pl.pallas_call(
        _fused_kernel,
        grid=(_NA,),
        in_specs=dense_specs + full3,
        out_specs=pl.BlockSpec((3, 8), lambda a: (0, 0), memory_space=pltpu.SMEM),
        out_shape=jax.ShapeDtypeStruct((3, 8), jnp.float32),
    )(*preds, *tmasks, psel_t, box_t, cls_r, mf_r)


def _prep_level(bt, bs, gh, gw, layer):
    nt = bt.shape[0]
    valid = jnp.isfinite(bt).all(1) & (bt[:, 1:5] > 0).all(1)
    av = jnp.asarray(_ANCHORS[layer])
    r = bt[:, 3:5][:, None, :] / av[None, :, :]
    j = jnp.maximum(r, 1.0 / r).max(2) < _ANCHOR_T
    mask = (valid[:, None] & j).reshape(-1)
    gx = bt[:, 2] * gw
    gy = bt[:, 3] * gh
    gi = jnp.clip(jnp.floor(gx).astype(jnp.int32), 0, gw - 1)
    gj = jnp.clip(jnp.floor(gy).astype(jnp.int32), 0, gh - 1)
    b = jnp.clip(bt[:, 0].astype(jnp.int32), 0, bs - 1)

    box_t = jnp.stack([
        jnp.broadcast_to((gx - gi)[:, None], (nt, _NA)),
        jnp.broadcast_to((gy - gj)[:, None], (nt, _NA)),
        bt[:, 3][:, None] / av[None, :, 0],
        bt[:, 4][:, None] / av[None, :, 1],
    ], axis=2).astype(jnp.float32).reshape(-1, 4)
    cls_t = jnp.clip(bt[:, 1].astype(jnp.int32), 0, _NC - 1)

    a_b = jnp.broadcast_to(jnp.arange(_NA, dtype=jnp.int32)[None, :], (nt, _NA)).reshape(-1)
    b_b = jnp.broadcast_to(b[:, None], (nt, _NA)).reshape(-1)
    gj_b = jnp.broadcast_to(gj[:, None], (nt, _NA)).reshape(-1)
    gi_b = jnp.broadcast_to(gi[:, None], (nt, _NA)).reshape(-1)
    cls_b = jnp.broadcast_to(cls_t[:, None], (nt, _NA)).reshape(-1)

    hw = gh * gw
    cell = ((b_b * _NA + a_b) * gh + gj_b) * gw + gi_b
    chan0 = (b_b * 36 + a_b * 12) * hw + gj_b * gw + gi_b
    idx = chan0[:, None] + (jnp.arange(12, dtype=jnp.int32) * hw)[None, :]

    mf = mask.astype(jnp.float32)
    big = bs * _NA * hw
    ids = jnp.where(mask, cell, big)
    tmask = (jnp.zeros((big,), jnp.float32)
             .at[ids].set(1.0, mode="drop")
             .reshape(bs, _NA, gh, gw))
    return idx, box_t, cls_b, mf, tmask


def _yolo_loss(p3, p4, p5, targets):
    preds = [p3, p4, p5]
    bt = targets.astype(jnp.float32)

    tmasks = []
    psel_ts, box_ts, cls_rs, mf_rs, inv_ns = [], [], [], [], []
    for i, pred in enumerate(preds):
        bs, _, gh, gw = pred.shape
        idx, box_t, cls_b, mf, tmask = _prep_level(bt, bs, gh, gw, i)
        psel = pred.reshape(-1)[idx]
        psel_ts.append(psel.T)
        box_ts.append(box_t.T)
        cls_rs.append(cls_b.astype(jnp.float32)[None, :])
        mf_rs.append(mf[None, :])
        inv_ns.append(1.0 / (bs * _NA * gh * gw))
        tmasks.append(tmask)

    out = _fused_call(preds, tmasks, jnp.stack(psel_ts), jnp.stack(box_ts),
                      jnp.stack(cls_rs), jnp.stack(mf_rs))
    box_sums, cls_sums, cnts = out[:, 0], out[:, 1], out[:, 2]
    obj_sums, corrs = out[:, 3], out[:, 4]

    safe = jnp.maximum(cnts, 1.0)
    has = (cnts > 0).astype(jnp.float32)
    nlay = jnp.maximum(has.sum(), 1.0)
    obj_l = (obj_sums - corrs) * jnp.asarray(inv_ns, jnp.float32)
    obj_l = obj_l * jnp.asarray(_BALANCE, jnp.float32)

    lbox = (jnp.sum((box_sums / safe) * has) / nlay) * _BOX_W
    lobj = jnp.mean(obj_l) * _OBJ_W
    lcls = (jnp.sum((cls_sums / (safe * _NC)) * has) / nlay) * _CLS_W
    loss = lbox + lobj + lcls
    return loss, lbox, lobj, lcls


_jitted = jax.jit(_yolo_loss)


def kernel(p3, p4, p5, targets):
    return _jitted(p3, p4, p5, targets)
